# baseline (speedup 1.0000x reference)
import functools

import jax
import jax.numpy as jnp
from jax import lax
from jax.experimental import pallas as pl
from jax.experimental.pallas import tpu as pltpu
from jax.experimental.pallas import tpu_sc as plsc

_B = 100000
_D = 256
_V = 95
_C = 128
_L = 16
_NW = 32
_NCHUNK = (_B + _C - 1) // _C
_LAST = _B - _C
_ITERS = -(-_NCHUNK // _NW)

_mesh = plsc.VectorSubcoreMesh(core_axis_name="c", subcore_axis_name="s")


@functools.partial(
    pl.kernel,
    mesh=_mesh,
    compiler_params=pltpu.CompilerParams(needs_layout_passes=False),
    out_type=jax.ShapeDtypeStruct((_B * _D,), jnp.float32),
    scratch_types=[
        pltpu.VMEM((_V * _D,), jnp.float32),
        pltpu.VMEM((_C,), jnp.int32),
        pltpu.VMEM((_C,), jnp.int32),
        pltpu.VMEM((_C,), jnp.int32),
        pltpu.VMEM((_C * _D,), jnp.float32),
        pltpu.VMEM((_C * _D,), jnp.float32),
        pltpu.VMEM((_C * _D,), jnp.float32),
        pltpu.SemaphoreType.DMA,
        pltpu.SemaphoreType.DMA,
        pltpu.SemaphoreType.DMA,
    ],
)
def _emb_lookup(idx_hbm, table_hbm, out_hbm,
                table_v, i0, i1, i2, r0, r1, r2,
                o0, o1, o2):
    w = lax.axis_index("s") * 2 + lax.axis_index("c")
    idx_bufs = (i0, i1, i2)
    row_bufs = (r0, r1, r2)
    osems = (o0, o1, o2)

    pltpu.sync_copy(table_hbm, table_v)

    lanes = lax.iota(jnp.int32, _L)

    def chunk_start(i):
        return jnp.minimum((w + _NW * i) * _C, _LAST)

    def fire_out(i, s):
        st = chunk_start(i)
        pltpu.async_copy(row_bufs[s], out_hbm.at[pl.ds(st * _D, _C * _D)],
                         osems[s])

    def wait_out(i, s):
        st = chunk_start(i)
        pltpu.make_async_copy(
            row_bufs[s], out_hbm.at[pl.ds(st * _D, _C * _D)], osems[s]).wait()

    def assemble(i, s):
        st = chunk_start(i)
        pltpu.sync_copy(idx_hbm.at[pl.ds(st, _C)], idx_bufs[s])

        def group(g, carry):
            gl = g * _L + lanes
            rows = plsc.load_gather(idx_bufs[s], [gl])
            src_base = rows * _D
            dst_base = gl * _D

            def colblock(co, carry2):
                cb = co * _L
                for ci in range(_L):
                    vals = plsc.load_gather(table_v, [src_base + (cb + ci)])
                    plsc.store_scatter(row_bufs[s], [dst_base + (cb + ci)],
                                       vals)
                return carry2

            lax.fori_loop(0, _D // _L, colblock, 0)
            return carry

        lax.fori_loop(0, _C // _L, group, 0)
        fire_out(i, s)

    def body(j, carry):
        for t in range(3):
            i = 3 * j + t

            @pl.when(j > 0)
            def _():
                wait_out(i - 3, t)

            assemble(i, t)
        return carry

    lax.fori_loop(0, (_ITERS - 1) // 3, body, 0)

    wait_out(_ITERS - 4, 0)
    assemble(_ITERS - 1, 0)
    wait_out(_ITERS - 3, 1)
    wait_out(_ITERS - 2, 2)
    wait_out(_ITERS - 1, 0)


def kernel(x, emb_weight):
    flat = _emb_lookup(x.astype(jnp.int32), emb_weight.reshape(-1))
    return flat.reshape(_B, _D)


# speedup vs baseline: 4.8766x; 4.8766x over previous
import functools

import jax
import jax.numpy as jnp
from jax import lax
from jax.experimental import pallas as pl
from jax.experimental.pallas import tpu as pltpu
from jax.experimental.pallas import tpu_sc as plsc

_B = 100000
_D = 256
_V = 95
_C = 128
_L = 16
_NW = 32
_LAST = _B - _C

_BLK = 512
_BT = 50176
_TV = 128

_B_SC = _B - _BT
_NCHUNK = (_B_SC + _C - 1) // _C
_ITERS = -(-_NCHUNK // _NW)

_mesh = plsc.VectorSubcoreMesh(core_axis_name="c", subcore_axis_name="s")


@functools.partial(
    pl.kernel,
    mesh=_mesh,
    compiler_params=pltpu.CompilerParams(needs_layout_passes=False),
    out_type=jax.ShapeDtypeStruct((_B * _D,), jnp.float32),
    scratch_types=[
        pltpu.VMEM((_V * _D,), jnp.float32),
        pltpu.VMEM((_ITERS * _C,), jnp.int32),
        pltpu.VMEM((_C * _D,), jnp.float32),
        pltpu.VMEM((_C * _D,), jnp.float32),
        pltpu.VMEM((_C * _D,), jnp.float32),
        pltpu.SemaphoreType.DMA,
        pltpu.SemaphoreType.DMA,
        pltpu.SemaphoreType.DMA,
        pltpu.SemaphoreType.DMA,
    ],
)
def _emb_lookup(idx_hbm, table_hbm, out_hbm,
                table_v, ibuf, r0, r1, r2,
                o0, o1, o2, isem):
    w = lax.axis_index("s") * 2 + lax.axis_index("c")
    row_bufs = (r0, r1, r2)
    osems = (o0, o1, o2)

    lanes = lax.iota(jnp.int32, _L)
    rots = [jnp.bitwise_and(lanes + k, _L - 1) for k in range(_L)]

    def chunk_start(i):
        return jnp.minimum(_BT + (w + _NW * i) * _C, _LAST)

    for k in range(_ITERS):
        pltpu.async_copy(idx_hbm.at[pl.ds(chunk_start(k), _C)],
                         ibuf.at[pl.ds(k * _C, _C)], isem)
    pltpu.sync_copy(table_hbm, table_v)
    pltpu.make_async_copy(
        idx_hbm.at[pl.ds(0, _ITERS * _C)], ibuf, isem).wait()

    def fire_out(i, s):
        st = chunk_start(i)
        pltpu.async_copy(row_bufs[s], out_hbm.at[pl.ds(st * _D, _C * _D)],
                         osems[s])

    def wait_out(i, s):
        st = chunk_start(i)
        pltpu.make_async_copy(
            row_bufs[s], out_hbm.at[pl.ds(st * _D, _C * _D)], osems[s]).wait()

    def assemble(i, s):
        def group(g, carry):
            gl = g * _L + lanes
            rows = plsc.load_gather(ibuf, [i * _C + gl])
            src_base = rows * _D
            dst_base = gl * _D

            @plsc.parallel_loop(0, _D // _L, unroll=4)
            def colblock(cb):
                sb = src_base + cb * _L
                db = dst_base + cb * _L
                for k in range(_L):
                    vals = plsc.load_gather(table_v, [sb + rots[k]])
                    plsc.store_scatter(row_bufs[s], [db + rots[k]], vals)

            return carry

        lax.fori_loop(0, _C // _L, group, 0)
        fire_out(i, s)

    def body(j, carry):
        for t in range(3):
            i = 3 * j + t

            @pl.when(j > 0)
            def _():
                wait_out(i - 3, t)

            assemble(i, t)
        return carry

    lax.fori_loop(0, (_ITERS - 1) // 3, body, 0)

    wait_out(_ITERS - 4, 0)
    assemble(_ITERS - 1, 0)
    wait_out(_ITERS - 3, 1)
    wait_out(_ITERS - 2, 2)
    wait_out(_ITERS - 1, 0)


def _tc_fill(x_ref, tbl_ref, outa_ref, out_ref):
    del outa_ref
    xb = x_ref[0, 0, :]
    oh = xb[:, None] == lax.broadcasted_iota(jnp.int32, (_BLK, _TV), 1)
    out_ref[...] = jnp.dot(oh.astype(jnp.float32), tbl_ref[...],
                           preferred_element_type=jnp.float32)


_tc_call = pl.pallas_call(
    _tc_fill,
    grid=(_BT // _BLK,),
    in_specs=[
        pl.BlockSpec((1, 1, _BLK), lambda i: (i, 0, 0)),
        pl.BlockSpec((_TV, _D), lambda i: (0, 0)),
        pl.BlockSpec(memory_space=pl.ANY),
    ],
    out_specs=pl.BlockSpec((_BLK, _D), lambda i: (i, 0)),
    out_shape=jax.ShapeDtypeStruct((_B, _D), jnp.float32),
    input_output_aliases={2: 0},
)


def kernel(x, emb_weight):
    xi = x.astype(jnp.int32)
    flat = _emb_lookup(xi, emb_weight.reshape(-1))
    out2d = flat.reshape(_B, _D)
    x_tc = xi[:_BT].reshape(_BT // _BLK, 1, _BLK)
    tbl_pad = jnp.zeros((_TV, _D), jnp.float32).at[:_V].set(emb_weight)
    return _tc_call(x_tc, tbl_pad, out2d)


# speedup vs baseline: 14.1076x; 2.8929x over previous
import functools

import jax
import jax.numpy as jnp
from jax import lax
from jax.experimental import pallas as pl
from jax.experimental.pallas import tpu as pltpu
from jax.experimental.pallas import tpu_sc as plsc

_B = 100000
_D = 256
_V = 95
_C = 128
_L = 16
_NW = 32
_LAST = _B - _C

_BLK = 512
_BT = 0
_TV = 128

_B_SC = _B - _BT
_NCHUNK = (_B_SC + _C - 1) // _C
_ITERS = -(-_NCHUNK // _NW)

_mesh = plsc.VectorSubcoreMesh(core_axis_name="c", subcore_axis_name="s")


@functools.partial(
    pl.kernel,
    mesh=_mesh,
    compiler_params=pltpu.CompilerParams(needs_layout_passes=False),
    out_type=jax.ShapeDtypeStruct((_B, _D), jnp.float32),
    scratch_types=[
        pltpu.VMEM((_V * _D,), jnp.float32),
        pltpu.VMEM((_ITERS * _C,), jnp.int32),
        pltpu.VMEM((_C, _D), jnp.float32),
        pltpu.VMEM((_C, _D), jnp.float32),
        pltpu.VMEM((_C, _D), jnp.float32),
        pltpu.SemaphoreType.DMA,
        pltpu.SemaphoreType.DMA,
        pltpu.SemaphoreType.DMA,
        pltpu.SemaphoreType.DMA,
    ],
)
def _emb_lookup(idx_hbm, table_hbm, out_hbm,
                table_v, ibuf, r0, r1, r2,
                o0, o1, o2, isem):
    w = lax.axis_index("s") * 2 + lax.axis_index("c")
    row_bufs = (r0, r1, r2)
    osems = (o0, o1, o2)

    lanes = lax.iota(jnp.int32, _L)
    rots = [jnp.bitwise_and(lanes + k, _L - 1) for k in range(_L)]

    def chunk_start(i):
        return jnp.minimum(_BT + (w + _NW * i) * _C, _LAST)

    for k in range(_ITERS):
        pltpu.async_copy(idx_hbm.at[pl.ds(chunk_start(k), _C)],
                         ibuf.at[pl.ds(k * _C, _C)], isem)
    pltpu.sync_copy(table_hbm, table_v)
    pltpu.make_async_copy(
        idx_hbm.at[pl.ds(0, _ITERS * _C)], ibuf, isem).wait()

    def fire_out(i, s):
        st = chunk_start(i)
        pltpu.async_copy(row_bufs[s], out_hbm.at[pl.ds(st, _C)], osems[s])

    def wait_out(i, s):
        st = chunk_start(i)
        pltpu.make_async_copy(
            row_bufs[s], out_hbm.at[pl.ds(st, _C)], osems[s]).wait()

    def assemble(i, s):
        def group(g, carry):
            gl = g * _L + lanes
            rows = plsc.load_gather(ibuf, [i * _C + gl])
            src_base = rows * _D

            @plsc.parallel_loop(0, _D // _L, unroll=4)
            def colblock(cb):
                sb = src_base + cb * _L
                cb16 = cb * _L
                for k in range(_L):
                    vals = plsc.load_gather(table_v, [sb + rots[k]])
                    plsc.store_scatter(row_bufs[s], [gl, cb16 + rots[k]],
                                       vals)

            return carry

        lax.fori_loop(0, _C // _L, group, 0)
        fire_out(i, s)

    def body(j, carry):
        for t in range(3):
            i = 3 * j + t

            @pl.when(j > 0)
            def _():
                wait_out(i - 3, t)

            assemble(i, t)
        return carry

    lax.fori_loop(0, (_ITERS - 1) // 3, body, 0)

    wait_out(_ITERS - 4, 0)
    assemble(_ITERS - 1, 0)
    wait_out(_ITERS - 3, 1)
    wait_out(_ITERS - 2, 2)
    wait_out(_ITERS - 1, 0)


def kernel(x, emb_weight):
    return _emb_lookup(x.astype(jnp.int32), emb_weight.reshape(-1))


# speedup vs baseline: 14.1126x; 1.0004x over previous
import functools

import jax
import jax.numpy as jnp
from jax import lax
from jax.experimental import pallas as pl
from jax.experimental.pallas import tpu as pltpu
from jax.experimental.pallas import tpu_sc as plsc

_B = 100000
_D = 256
_V = 95
_C = 128
_L = 16
_NW = 32
_LAST = _B - _C
_NCHUNK = (_B + _C - 1) // _C
_ITERS = -(-_NCHUNK // _NW)

_mesh = plsc.VectorSubcoreMesh(core_axis_name="c", subcore_axis_name="s")


@functools.partial(
    pl.kernel,
    mesh=_mesh,
    compiler_params=pltpu.CompilerParams(needs_layout_passes=False),
    out_type=jax.ShapeDtypeStruct((_B, _D), jnp.float32),
    scratch_types=[
        pltpu.VMEM((_V * _D,), jnp.float32),
        pltpu.VMEM((_ITERS * _C,), jnp.int32),
        pltpu.VMEM((_C, _D), jnp.float32),
        pltpu.VMEM((_C, _D), jnp.float32),
        pltpu.VMEM((_C, _D), jnp.float32),
        pltpu.SemaphoreType.DMA,
        pltpu.SemaphoreType.DMA,
        pltpu.SemaphoreType.DMA,
        pltpu.SemaphoreType.DMA,
    ],
)
def _emb_lookup(idx_hbm, table_hbm, out_hbm,
                table_v, ibuf, r0, r1, r2,
                o0, o1, o2, isem):
    w = lax.axis_index("s") * 2 + lax.axis_index("c")
    row_bufs = (r0, r1, r2)
    osems = (o0, o1, o2)

    lanes = lax.iota(jnp.int32, _L)
    rots = [jnp.bitwise_and(lanes + k, _L - 1) for k in range(_L)]

    def chunk_start(i):
        return jnp.minimum((w + _NW * i) * _C, _LAST)

    for k in range(_ITERS):
        pltpu.async_copy(idx_hbm.at[pl.ds(chunk_start(k), _C)],
                         ibuf.at[pl.ds(k * _C, _C)], isem)
    pltpu.sync_copy(table_hbm, table_v)
    pltpu.make_async_copy(
        idx_hbm.at[pl.ds(0, _ITERS * _C)], ibuf, isem).wait()

    def fire_out(i, s):
        st = chunk_start(i)
        pltpu.async_copy(row_bufs[s], out_hbm.at[pl.ds(st, _C)], osems[s])

    def wait_out(i, s):
        st = chunk_start(i)
        pltpu.make_async_copy(
            row_bufs[s], out_hbm.at[pl.ds(st, _C)], osems[s]).wait()

    def assemble(i, s):
        def group(g, carry):
            gl = g * _L + lanes
            rows = plsc.load_gather(ibuf, [i * _C + gl])
            src_base = rows * _D

            @plsc.parallel_loop(0, _D // _L, unroll=4)
            def colblock(cb):
                sb = src_base + cb * _L
                cb16 = cb * _L
                for k in range(_L):
                    vals = plsc.load_gather(table_v, [sb + rots[k]])
                    plsc.store_scatter(row_bufs[s], [gl, cb16 + rots[k]],
                                       vals)

            return carry

        lax.fori_loop(0, _C // _L, group, 0)
        fire_out(i, s)

    def body(j, carry):
        for t in range(3):
            i = 3 * j + t

            @pl.when(j > 0)
            def _():
                wait_out(i - 3, t)

            assemble(i, t)
        return carry

    lax.fori_loop(0, (_ITERS - 1) // 3, body, 0)

    wait_out(_ITERS - 4, 0)
    assemble(_ITERS - 1, 0)
    wait_out(_ITERS - 3, 1)
    wait_out(_ITERS - 2, 2)
    wait_out(_ITERS - 1, 0)


def kernel(x, emb_weight):
    return _emb_lookup(x.astype(jnp.int32), emb_weight.reshape(-1))
